# baseline (speedup 1.0000x reference)
import functools

import jax
import jax.numpy as jnp
from jax import lax
from jax.experimental import pallas as pl
from jax.experimental.pallas import tpu as pltpu
from jax.experimental.pallas import tpu_sc as plsc

_K = 16
_UN = 8


def _col_group_pass(buf, g, n):
    def max_step(i, c):
        ms, ixs = c
        base = i * _UN
        nms, nixs = [], []
        for r in range(_UN):
            v = buf[base + r, pl.ds(g * 16, 16)]
            gt = v > ms[r]
            nms.append(jnp.where(gt, v, ms[r]))
            nixs.append(jnp.where(gt, base + r, ixs[r]))
        return (tuple(nms), tuple(nixs))

    m0 = tuple(jnp.full((16,), -jnp.inf, jnp.float32) for _ in range(_UN))
    i0 = tuple(jnp.zeros((16,), jnp.int32) for _ in range(_UN))
    ms, ixs = lax.fori_loop(0, n // _UN, max_step, (m0, i0))
    M, IX = ms[0], ixs[0]
    for r in range(1, _UN):
        better = (ms[r] > M) | ((ms[r] == M) & (ixs[r] < IX))
        M = jnp.where(better, ms[r], M)
        IX = jnp.where(better, ixs[r], IX)

    def exp_step(i, ss):
        base = i * _UN
        return tuple(
            ss[r] + jnp.exp(buf[base + r, pl.ds(g * 16, 16)] - M)
            for r in range(_UN))

    s0 = tuple(jnp.zeros((16,), jnp.float32) for _ in range(_UN))
    ss = lax.fori_loop(0, n // _UN, exp_step, s0)
    s = ss[0]
    for r in range(1, _UN):
        s = s + ss[r]
    return IX, 1.0 / s


def _sc_logits(position_logits):
    b, n, _ = position_logits.shape
    full_per_w = _K * 4 // 32
    mesh = plsc.VectorSubcoreMesh(core_axis_name="c", subcore_axis_name="s")

    @functools.partial(
        pl.kernel,
        mesh=mesh,
        out_type=(
            jax.ShapeDtypeStruct((_K, n), jnp.int32),
            jax.ShapeDtypeStruct((_K, n), jnp.float32),
        ),
        scratch_types=[
            pltpu.VMEM((n, 128), jnp.float32),
            pltpu.VMEM((128,), jnp.int32),
            pltpu.VMEM((128,), jnp.float32),
        ],
    )
    def k(logits_hbm, preds_hbm, conf_hbm, buf, pvec, cvec):
        w = lax.axis_index("s") * 2 + lax.axis_index("c")
        for ft in range(full_per_w):
            t = w * full_per_w + ft
            bi = t // 4
            cb = t % 4
            pltpu.sync_copy(logits_hbm.at[bi, :, pl.ds(cb * 128, 128)], buf)
            for g in range(8):
                IX, C = _col_group_pass(buf, g, n)
                pvec[pl.ds(g * 16, 16)] = IX
                cvec[pl.ds(g * 16, 16)] = C
            pltpu.sync_copy(pvec, preds_hbm.at[bi, pl.ds(cb * 128, 128)])
            pltpu.sync_copy(cvec, conf_hbm.at[bi, pl.ds(cb * 128, 128)])

    return k(position_logits)


def _tail_body(logits_ref, preds_ref, conf_ref):
    n = logits_ref.shape[1]
    L = logits_ref[0]
    m = jnp.max(L, axis=0)
    ii = lax.broadcasted_iota(jnp.int32, L.shape, 0)
    t = L - m[None, :]
    preds_ref[0, 0] = jnp.min(jnp.where(t == 0.0, ii, n), axis=0)
    conf_ref[0, 0] = 1.0 / jnp.sum(jnp.exp(t), axis=0)


def _mt_from_preds(preds, n):
    ii = lax.broadcasted_iota(jnp.int32, (n, n), 0)
    pp = lax.broadcasted_iota(jnp.int32, (n, n), 1)
    winner = jnp.max(jnp.where(preds[:, None] == pp, ii, -1), axis=0)
    jj16 = lax.broadcasted_iota(jnp.int16, (n, n), 0)
    one = jnp.bfloat16(1.0)
    zero = jnp.bfloat16(0.0)
    Pt = jnp.where(jj16 == winner[None, :].astype(jnp.int16), one, zero)
    inner = (Pt[:, :-2] + Pt[:, 1:-1] + Pt[:, 2:]) * jnp.bfloat16(1.0 / 3.0)
    return jnp.concatenate([Pt[:, :1], inner, Pt[:, -1:]], axis=1)


def _dot_out(feat, Mt):
    return lax.dot_general(
        feat.astype(jnp.bfloat16), Mt,
        dimension_numbers=(((0,), (0,)), ((), ())),
        preferred_element_type=jnp.float32,
    )


def _fused_body(logits_ref, feat_ref, out_ref, conf_ref):
    n = logits_ref.shape[1]
    L = logits_ref[0]
    m = jnp.max(L, axis=0)
    ii = lax.broadcasted_iota(jnp.int32, (n, n), 0)
    t = L - m[None, :]
    preds = jnp.min(jnp.where(t == 0.0, ii, n), axis=0)
    conf_ref[0, 0] = 1.0 / jnp.sum(jnp.exp(t), axis=0)
    out_ref[0] = _dot_out(feat_ref[0], _mt_from_preds(preds, n))


def _recon_body(_, preds_ref, feat_ref, out_ref):
    n = feat_ref.shape[1]
    preds = preds_ref[0, 0]
    out_ref[0] = _dot_out(feat_ref[0], _mt_from_preds(preds, n))


def kernel(features, position_logits):
    b, n, d = features.shape
    preds_sc, conf_sc = _sc_logits(position_logits)
    preds_t, conf_t = pl.pallas_call(
        _tail_body,
        grid=(_K,),
        in_specs=[pl.BlockSpec((1, n, 128), lambda i: (i, 0, 4))],
        out_specs=[
            pl.BlockSpec((1, 1, 128), lambda i: (i, 0, 0)),
            pl.BlockSpec((1, 1, 128), lambda i: (i, 0, 0)),
        ],
        out_shape=[
            jax.ShapeDtypeStruct((_K, 1, 128), jnp.int32),
            jax.ShapeDtypeStruct((_K, 1, 128), jnp.float32),
        ],
    )(position_logits)
    preds_sc = jnp.concatenate(
        [preds_sc[:, :512], preds_t[:, 0, : n - 512]], axis=1)
    conf_sc = jnp.concatenate(
        [conf_sc[:, :512], conf_t[:, 0, : n - 512]], axis=1)
    recon_t, conf3 = pl.pallas_call(
        _fused_body,
        grid=(b - _K,),
        in_specs=[
            pl.BlockSpec((1, n, n), lambda i: (i + _K, 0, 0)),
            pl.BlockSpec((1, n, d), lambda i: (i + _K, 0, 0)),
        ],
        out_specs=[
            pl.BlockSpec((1, d, n), lambda i: (i + _K, 0, 0)),
            pl.BlockSpec((1, 1, n), lambda i: (i + _K, 0, 0)),
        ],
        out_shape=[
            jax.ShapeDtypeStruct((b, d, n), jnp.float32),
            jax.ShapeDtypeStruct((b, 1, n), jnp.float32),
        ],
    )(position_logits, features)
    recon_t = pl.pallas_call(
        _recon_body,
        grid=(_K,),
        in_specs=[
            pl.BlockSpec(memory_space=pl.ANY),
            pl.BlockSpec((1, 1, n), lambda i: (i, 0, 0)),
            pl.BlockSpec((1, n, d), lambda i: (i, 0, 0)),
        ],
        out_specs=pl.BlockSpec((1, d, n), lambda i: (i, 0, 0)),
        out_shape=jax.ShapeDtypeStruct((b, d, n), jnp.float32),
        input_output_aliases={0: 0},
    )(recon_t, preds_sc.reshape(_K, 1, n), features)
    g = int(round(n ** 0.5))
    conf = jnp.concatenate([conf_sc, conf3[_K:, 0, :]], axis=0)
    return (recon_t.reshape(b, d, g, g), conf)


# speedup vs baseline: 1.3044x; 1.3044x over previous
import jax
import jax.numpy as jnp
from jax import lax
from jax.experimental import pallas as pl


def _one(logits_ref, feat_ref, out_ref, conf_ref, k):
    n = logits_ref.shape[1]
    L = logits_ref[k]
    m = jnp.max(L, axis=0)
    ii = lax.broadcasted_iota(jnp.int32, (n, n), 0)
    t = L - m[None, :]
    preds = jnp.min(jnp.where(t == 0.0, ii, n), axis=0)
    s = jnp.sum(jnp.exp(t), axis=0)
    conf_ref[k, 0] = 1.0 / s

    pp = lax.broadcasted_iota(jnp.int32, (n, n), 1)
    hit = preds[:, None] == pp
    winner = jnp.max(jnp.where(hit, ii, -1), axis=0)
    jj16 = lax.broadcasted_iota(jnp.int16, (n, n), 0)
    one = jnp.bfloat16(1.0)
    zero = jnp.bfloat16(0.0)
    Pt = jnp.where(jj16 == winner[None, :].astype(jnp.int16), one, zero)

    inner = (Pt[:, :-2] + Pt[:, 1:-1] + Pt[:, 2:]) * jnp.bfloat16(1.0 / 3.0)
    Mt = jnp.concatenate([Pt[:, :1], inner, Pt[:, -1:]], axis=1)

    out_ref[k] = lax.dot_general(
        feat_ref[k].astype(jnp.bfloat16), Mt,
        dimension_numbers=(((0,), (0,)), ((), ())),
        preferred_element_type=jnp.float32,
    )


def _body(logits_ref, feat_ref, out_ref, conf_ref):
    for k in range(logits_ref.shape[0]):
        _one(logits_ref, feat_ref, out_ref, conf_ref, k)


def kernel(features, position_logits):
    b, n, d = features.shape
    bb = 2
    recon_t, conf3 = pl.pallas_call(
        _body,
        grid=(b // bb,),
        in_specs=[
            pl.BlockSpec((bb, n, n), lambda i: (i, 0, 0)),
            pl.BlockSpec((bb, n, d), lambda i: (i, 0, 0)),
        ],
        out_specs=[
            pl.BlockSpec((bb, d, n), lambda i: (i, 0, 0)),
            pl.BlockSpec((bb, 1, n), lambda i: (i, 0, 0)),
        ],
        out_shape=[
            jax.ShapeDtypeStruct((b, d, n), jnp.float32),
            jax.ShapeDtypeStruct((b, 1, n), jnp.float32),
        ],
    )(position_logits, features)
    g = int(round(n ** 0.5))
    return (recon_t.reshape(b, d, g, g), conf3.reshape(b, n))


# speedup vs baseline: 1.3282x; 1.0183x over previous
import jax
import jax.numpy as jnp
from jax import lax
from jax.experimental import pallas as pl


def _one(logits_ref, feat_ref, out_ref, conf_ref, k):
    n = logits_ref.shape[1]
    L = logits_ref[k]
    m = jnp.max(L, axis=0)
    ii = lax.broadcasted_iota(jnp.int32, (n, n), 0)
    t = L - m[None, :]
    preds = jnp.min(jnp.where(t == 0.0, ii, n), axis=0)
    s = jnp.sum(jnp.exp(t), axis=0)
    conf_ref[k, 0] = 1.0 / s

    pp = lax.broadcasted_iota(jnp.int32, (n, n), 1)
    hit = preds[:, None] == pp
    winner = jnp.max(jnp.where(hit, ii, -1), axis=0)
    jj16 = lax.broadcasted_iota(jnp.int16, (n, n), 0)
    one = jnp.bfloat16(1.0)
    zero = jnp.bfloat16(0.0)
    Pt = jnp.where(jj16 == winner[None, :].astype(jnp.int16), one, zero)

    inner = (Pt[:, :-2] + Pt[:, 1:-1] + Pt[:, 2:]) * jnp.bfloat16(1.0 / 3.0)
    Mt = jnp.concatenate([Pt[:, :1], inner, Pt[:, -1:]], axis=1)

    out_ref[k] = lax.dot_general(
        feat_ref[k].astype(jnp.bfloat16), Mt,
        dimension_numbers=(((0,), (0,)), ((), ())),
        preferred_element_type=jnp.float32,
    )


def _body(logits_ref, feat_ref, out_ref, conf_ref):
    for k in range(logits_ref.shape[0]):
        _one(logits_ref, feat_ref, out_ref, conf_ref, k)


def kernel(features, position_logits):
    b, n, d = features.shape
    bb = 4
    recon_t, conf3 = pl.pallas_call(
        _body,
        grid=(b // bb,),
        in_specs=[
            pl.BlockSpec((bb, n, n), lambda i: (i, 0, 0)),
            pl.BlockSpec((bb, n, d), lambda i: (i, 0, 0)),
        ],
        out_specs=[
            pl.BlockSpec((bb, d, n), lambda i: (i, 0, 0)),
            pl.BlockSpec((bb, 1, n), lambda i: (i, 0, 0)),
        ],
        out_shape=[
            jax.ShapeDtypeStruct((b, d, n), jnp.float32),
            jax.ShapeDtypeStruct((b, 1, n), jnp.float32),
        ],
    )(position_logits, features)
    g = int(round(n ** 0.5))
    return (recon_t.reshape(b, d, g, g), conf3.reshape(b, n))
